# baseline (speedup 1.0000x reference)
import functools

import jax
import jax.numpy as jnp
from jax import lax
from jax.experimental import pallas as pl
from jax.experimental.pallas import tpu as pltpu
from jax.experimental.pallas import tpu_sc as plsc

N_NODES = 10000
N_EDGES = 320000
D_FEAT = 128

NC = 2
NS = 16
NW = NC * NS

EPT = N_EDGES // NW
DEG_ITERS = EPT // 16

K = 64
CH = 1024
NCH = 10
PAD_EPT = NCH * CH
PAD_EDGES = PAD_EPT * NW

LC = 3072
ACC_ROWS = 2560
DUMMY_DST = 2544
PQ = (0, 2504, 5008, 7512, 10000)

HST = 624
OCP = 152

_mesh = plsc.VectorSubcoreMesh(core_axis_name="c", subcore_axis_name="s")
_sc_params = pltpu.CompilerParams(needs_layout_passes=False)


@functools.partial(
    pl.kernel,
    out_type=jax.ShapeDtypeStruct((NW * N_NODES,), jnp.float32),
    mesh=_mesh,
    compiler_params=_sc_params,
    scratch_types=[
        pltpu.VMEM((EPT,), jnp.int32),
        pltpu.VMEM((N_NODES,), jnp.float32),
    ],
)
def _deg(src_hbm, out_hbm, src_v, hist_v):
    c = lax.axis_index("c")
    s = lax.axis_index("s")
    wid = s * NC + c
    pltpu.sync_copy(src_hbm.at[pl.ds(wid * EPT, EPT)], src_v)

    def _zero(i, carry):
        hist_v[pl.ds(i * 16, 16)] = jnp.zeros((16,), jnp.float32)
        return carry

    lax.fori_loop(0, N_NODES // 16, _zero, 0)

    ones = jnp.ones((16,), jnp.float32)

    def _accum(i, carry):
        idx = src_v[pl.ds(i * 16, 16)]
        plsc.addupdate_scatter(hist_v, [idx], ones)
        return carry

    lax.fori_loop(0, DEG_ITERS, _accum, 0)
    pltpu.sync_copy(hist_v, out_hbm.at[pl.ds(wid * N_NODES, N_NODES)])


def _prescale_body(pt_ref, feat_ref, h_ref, norm_ref):
    deg = jnp.sum(pt_ref[...], axis=1, keepdims=True)
    norm = jnp.where(deg > 0.0, lax.rsqrt(jnp.maximum(deg, 1e-12)), 0.0)
    norm_ref[...] = norm
    h_ref[...] = feat_ref[...] * norm


_prescale = pl.pallas_call(
    _prescale_body,
    out_shape=(
        jax.ShapeDtypeStruct((N_NODES, D_FEAT), jnp.float32),
        jax.ShapeDtypeStruct((N_NODES, 1), jnp.float32),
    ),
)


@functools.partial(
    pl.kernel,
    out_type=jax.ShapeDtypeStruct((NC, N_NODES, D_FEAT), jnp.float32),
    mesh=_mesh,
    compiler_params=_sc_params,
    scratch_types=[
        pltpu.VMEM((CH,), jnp.int32),
        pltpu.VMEM((CH,), jnp.int32),
        pltpu.VMEM((CH,), jnp.int32),
        pltpu.VMEM((CH,), jnp.int32),
        pltpu.VMEM((LC,), jnp.int32),
        pltpu.VMEM((LC,), jnp.int32),
        pltpu.VMEM((K,), jnp.int32),
        pltpu.VMEM((K, D_FEAT), jnp.float32),
        pltpu.VMEM_SHARED((N_NODES, D_FEAT), jnp.float32),
        pltpu.VMEM_SHARED((ACC_ROWS, D_FEAT), jnp.float32),
        pltpu.SemaphoreType.DMA,
        pltpu.SemaphoreType.DMA,
        pltpu.SemaphoreType.DMA,
    ],
)
def _agg(h_hbm, src_hbm, dst_hbm, out_hbm, chs0, chs1, chd0, chd1,
         cs, cd, dbuf, buf, h_sp, acc, sem, csem0, csem1):
    c = lax.axis_index("c")
    s = lax.axis_index("s")
    wid = s * NC + c
    chs = (chs0, chs1)
    chd = (chd0, chd1)
    csem = (csem0, csem1)

    pltpu.sync_copy(h_hbm.at[pl.ds(s * HST, HST)], h_sp.at[pl.ds(s * HST, HST)])

    @pl.when(s == NS - 1)
    def _hst_tail():
        pltpu.sync_copy(h_hbm.at[pl.ds(NS * HST, N_NODES - NS * HST)],
                        h_sp.at[pl.ds(NS * HST, N_NODES - NS * HST)])

    ebase = wid * PAD_EPT

    def _chunk_start(ch, q):
        pltpu.async_copy(src_hbm.at[pl.ds(ebase + ch * CH, CH)], chs[q],
                         csem[q])
        pltpu.async_copy(dst_hbm.at[pl.ds(ebase + ch * CH, CH)], chd[q],
                         csem[q])

    def _chunk_wait(q):
        pltpu.make_async_copy(src_hbm.at[pl.ds(0, CH)], chs[q],
                              csem[q]).wait()
        pltpu.make_async_copy(dst_hbm.at[pl.ds(0, CH)], chd[q],
                              csem[q]).wait()

    for p in range(4):
        lo = PQ[p]
        hi = PQ[p + 1]
        psz = hi - lo

        def _zbuf(i, carry):
            for j in range(D_FEAT // 16):
                buf[i, pl.ds(j * 16, 16)] = jnp.zeros((16,), jnp.float32)
            return carry

        lax.fori_loop(0, K, _zbuf, 0)
        zb = s * (ACC_ROWS // NS)
        for z in range(ACC_ROWS // NS // K):
            pltpu.sync_copy(buf, acc.at[pl.ds(zb + z * K, K)])
        zt = ACC_ROWS // NS - (ACC_ROWS // NS // K) * K
        pltpu.sync_copy(buf.at[pl.ds(0, zt)],
                        acc.at[pl.ds(zb + (ACC_ROWS // NS // K) * K, zt)])

        def _fill(i, carry):
            cs[pl.ds(i * 16, 16)] = jnp.zeros((16,), jnp.int32)
            cd[pl.ds(i * 16, 16)] = jnp.full((16,), DUMMY_DST, jnp.int32)
            return carry

        lax.fori_loop(0, LC // 16, _fill, 0)
        plsc.subcore_barrier()

        _chunk_start(0, 0)
        ptr = jnp.int32(0)
        for ch in range(NCH):
            q = ch % 2
            _chunk_wait(q)
            if ch + 1 < NCH:
                _chunk_start(ch + 1, (ch + 1) % 2)

            def _group(g, ptr_c):
                sv = chs[q][pl.ds(g * 16, 16)]
                dv = chd[q][pl.ds(g * 16, 16)]
                m = (dv >= lo) & (dv < hi)
                mi = m.astype(jnp.int32)
                pos = ptr_c + plsc.cumsum(mi) - 1
                plsc.store_scatter(cs, [pos], sv, mask=m)
                plsc.store_scatter(cd, [pos], dv - lo, mask=m)
                return ptr_c + jnp.sum(mi)

            ptr = lax.fori_loop(0, CH // 16, _group, ptr)

        nb = (ptr + (K - 1)) // K

        def _batch(j, carry):
            pltpu.async_copy(h_sp.at[cs.at[pl.ds(j * K, K)]], buf,
                             sem).wait()
            for i in range(K // 16):
                dbuf[pl.ds(i * 16, 16)] = cd[pl.ds(j * K + i * 16, 16)]
            pltpu.sync_copy(buf, acc.at[dbuf], add=True)
            return carry

        lax.fori_loop(0, nb, _batch, 0)
        plsc.subcore_barrier()

        pltpu.sync_copy(acc.at[pl.ds(s * OCP, OCP)],
                        out_hbm.at[c, pl.ds(lo + s * OCP, OCP)])

        @pl.when(s == NS - 1)
        def _ocp_tail():
            tail = psz - NS * OCP
            pltpu.sync_copy(acc.at[pl.ds(NS * OCP, tail)],
                            out_hbm.at[c, pl.ds(lo + NS * OCP, tail)])

        plsc.subcore_barrier()


def _combine_body(p_ref, norm_ref, o_ref):
    o_ref[...] = (p_ref[0] + p_ref[1]) * norm_ref[...]


_combine = pl.pallas_call(
    _combine_body,
    out_shape=jax.ShapeDtypeStruct((N_NODES, D_FEAT), jnp.float32),
)


def kernel(features, edge_index):
    src = edge_index[0]
    dst = edge_index[1]

    partials = _deg(src).reshape(NW, N_NODES)
    h, norm = _prescale(partials.T, features)

    pad = PAD_EDGES - N_EDGES
    src_p = jnp.concatenate([src, jnp.zeros((pad,), jnp.int32)])
    dst_p = jnp.concatenate([dst, jnp.full((pad,), N_NODES, jnp.int32)])

    p2 = _agg(h, src_p, dst_p)
    return _combine(p2, norm)


# speedup vs baseline: 1.0233x; 1.0233x over previous
import functools

import jax
import jax.numpy as jnp
from jax import lax
from jax.experimental import pallas as pl
from jax.experimental.pallas import tpu as pltpu
from jax.experimental.pallas import tpu_sc as plsc

N_NODES = 10000
N_EDGES = 320000
D_FEAT = 128

NC = 2
NS = 16
NW = NC * NS

EPT = N_EDGES // NW
DEG_ITERS = EPT // 16

K = 128
CH = 1024
NCH = 10
PAD_EPT = NCH * CH
PAD_EDGES = PAD_EPT * NW

LC = 3072
ACC_ROWS = 2560
DUMMY_DST = 2544
PQ = (0, 2504, 5008, 7512, 10000)

HST = 624
OCP = 152

_mesh = plsc.VectorSubcoreMesh(core_axis_name="c", subcore_axis_name="s")
_sc_params = pltpu.CompilerParams(needs_layout_passes=False)


@functools.partial(
    pl.kernel,
    out_type=jax.ShapeDtypeStruct((NW * N_NODES,), jnp.float32),
    mesh=_mesh,
    compiler_params=_sc_params,
    scratch_types=[
        pltpu.VMEM((EPT,), jnp.int32),
        pltpu.VMEM((N_NODES,), jnp.float32),
    ],
)
def _deg(src_hbm, out_hbm, src_v, hist_v):
    c = lax.axis_index("c")
    s = lax.axis_index("s")
    wid = s * NC + c
    pltpu.sync_copy(src_hbm.at[pl.ds(wid * EPT, EPT)], src_v)

    def _zero(i, carry):
        hist_v[pl.ds(i * 16, 16)] = jnp.zeros((16,), jnp.float32)
        return carry

    lax.fori_loop(0, N_NODES // 16, _zero, 0)

    ones = jnp.ones((16,), jnp.float32)

    def _accum(i, carry):
        idx = src_v[pl.ds(i * 16, 16)]
        plsc.addupdate_scatter(hist_v, [idx], ones)
        return carry

    lax.fori_loop(0, DEG_ITERS, _accum, 0)
    pltpu.sync_copy(hist_v, out_hbm.at[pl.ds(wid * N_NODES, N_NODES)])


def _prescale_body(pt_ref, feat_ref, h_ref, norm_ref):
    deg = jnp.sum(pt_ref[...], axis=1, keepdims=True)
    norm = jnp.where(deg > 0.0, lax.rsqrt(jnp.maximum(deg, 1e-12)), 0.0)
    norm_ref[...] = norm
    h_ref[...] = feat_ref[...] * norm


_prescale = pl.pallas_call(
    _prescale_body,
    out_shape=(
        jax.ShapeDtypeStruct((N_NODES, D_FEAT), jnp.float32),
        jax.ShapeDtypeStruct((N_NODES, 1), jnp.float32),
    ),
)


@functools.partial(
    pl.kernel,
    out_type=jax.ShapeDtypeStruct((NC, N_NODES, D_FEAT), jnp.float32),
    mesh=_mesh,
    compiler_params=_sc_params,
    scratch_types=[
        pltpu.VMEM((CH,), jnp.int32),
        pltpu.VMEM((CH,), jnp.int32),
        pltpu.VMEM((CH,), jnp.int32),
        pltpu.VMEM((CH,), jnp.int32),
        pltpu.VMEM((LC,), jnp.int32),
        pltpu.VMEM((LC,), jnp.int32),
        pltpu.VMEM((K,), jnp.int32),
        pltpu.VMEM((K, D_FEAT), jnp.float32),
        pltpu.VMEM_SHARED((N_NODES, D_FEAT), jnp.float32),
        pltpu.VMEM_SHARED((ACC_ROWS, D_FEAT), jnp.float32),
        pltpu.SemaphoreType.DMA,
        pltpu.SemaphoreType.DMA,
        pltpu.SemaphoreType.DMA,
    ],
)
def _agg(h_hbm, src_hbm, dst_hbm, out_hbm, chs0, chs1, chd0, chd1,
         cs, cd, dbuf, buf, h_sp, acc, sem, csem0, csem1):
    c = lax.axis_index("c")
    s = lax.axis_index("s")
    wid = s * NC + c
    chs = (chs0, chs1)
    chd = (chd0, chd1)
    csem = (csem0, csem1)

    pltpu.sync_copy(h_hbm.at[pl.ds(s * HST, HST)], h_sp.at[pl.ds(s * HST, HST)])

    @pl.when(s == NS - 1)
    def _hst_tail():
        pltpu.sync_copy(h_hbm.at[pl.ds(NS * HST, N_NODES - NS * HST)],
                        h_sp.at[pl.ds(NS * HST, N_NODES - NS * HST)])

    ebase = wid * PAD_EPT

    def _chunk_start(ch, q):
        pltpu.async_copy(src_hbm.at[pl.ds(ebase + ch * CH, CH)], chs[q],
                         csem[q])
        pltpu.async_copy(dst_hbm.at[pl.ds(ebase + ch * CH, CH)], chd[q],
                         csem[q])

    def _chunk_wait(q):
        pltpu.make_async_copy(src_hbm.at[pl.ds(0, CH)], chs[q],
                              csem[q]).wait()
        pltpu.make_async_copy(dst_hbm.at[pl.ds(0, CH)], chd[q],
                              csem[q]).wait()

    for p in range(4):
        lo = PQ[p]
        hi = PQ[p + 1]
        psz = hi - lo

        def _zbuf(i, carry):
            for j in range(D_FEAT // 16):
                buf[i, pl.ds(j * 16, 16)] = jnp.zeros((16,), jnp.float32)
            return carry

        lax.fori_loop(0, K, _zbuf, 0)
        zb = s * (ACC_ROWS // NS)
        for z in range(ACC_ROWS // NS // K):
            pltpu.sync_copy(buf, acc.at[pl.ds(zb + z * K, K)])
        zt = ACC_ROWS // NS - (ACC_ROWS // NS // K) * K
        pltpu.sync_copy(buf.at[pl.ds(0, zt)],
                        acc.at[pl.ds(zb + (ACC_ROWS // NS // K) * K, zt)])

        def _fill(i, carry):
            cs[pl.ds(i * 16, 16)] = jnp.zeros((16,), jnp.int32)
            cd[pl.ds(i * 16, 16)] = jnp.full((16,), DUMMY_DST, jnp.int32)
            return carry

        lax.fori_loop(0, LC // 16, _fill, 0)
        plsc.subcore_barrier()

        _chunk_start(0, 0)
        ptr = jnp.int32(0)
        for ch in range(NCH):
            q = ch % 2
            _chunk_wait(q)
            if ch + 1 < NCH:
                _chunk_start(ch + 1, (ch + 1) % 2)

            def _group(g, ptr_c):
                sv = chs[q][pl.ds(g * 16, 16)]
                dv = chd[q][pl.ds(g * 16, 16)]
                m = (dv >= lo) & (dv < hi)
                mi = m.astype(jnp.int32)
                pos = ptr_c + plsc.cumsum(mi) - 1
                plsc.store_scatter(cs, [pos], sv, mask=m)
                plsc.store_scatter(cd, [pos], dv - lo, mask=m)
                return ptr_c + jnp.sum(mi)

            ptr = lax.fori_loop(0, CH // 16, _group, ptr)

        nb = (ptr + (K - 1)) // K

        def _batch(j, carry):
            pltpu.async_copy(h_sp.at[cs.at[pl.ds(j * K, K)]], buf,
                             sem).wait()
            for i in range(K // 16):
                dbuf[pl.ds(i * 16, 16)] = cd[pl.ds(j * K + i * 16, 16)]
            pltpu.sync_copy(buf, acc.at[dbuf], add=True)
            return carry

        lax.fori_loop(0, nb, _batch, 0)
        plsc.subcore_barrier()

        pltpu.sync_copy(acc.at[pl.ds(s * OCP, OCP)],
                        out_hbm.at[c, pl.ds(lo + s * OCP, OCP)])

        @pl.when(s == NS - 1)
        def _ocp_tail():
            tail = psz - NS * OCP
            pltpu.sync_copy(acc.at[pl.ds(NS * OCP, tail)],
                            out_hbm.at[c, pl.ds(lo + NS * OCP, tail)])

        plsc.subcore_barrier()


def _combine_body(p_ref, norm_ref, o_ref):
    o_ref[...] = (p_ref[0] + p_ref[1]) * norm_ref[...]


_combine = pl.pallas_call(
    _combine_body,
    out_shape=jax.ShapeDtypeStruct((N_NODES, D_FEAT), jnp.float32),
)


def kernel(features, edge_index):
    src = edge_index[0]
    dst = edge_index[1]

    partials = _deg(src).reshape(NW, N_NODES)
    h, norm = _prescale(partials.T, features)

    pad = PAD_EDGES - N_EDGES
    src_p = jnp.concatenate([src, jnp.zeros((pad,), jnp.int32)])
    dst_p = jnp.concatenate([dst, jnp.full((pad,), N_NODES, jnp.int32)])

    p2 = _agg(h, src_p, dst_p)
    return _combine(p2, norm)


# speedup vs baseline: 1.1290x; 1.1033x over previous
import functools

import jax
import jax.numpy as jnp
from jax import lax
from jax.experimental import pallas as pl
from jax.experimental.pallas import tpu as pltpu
from jax.experimental.pallas import tpu_sc as plsc

N_NODES = 10000
N_EDGES = 320000
D_FEAT = 128

NC = 2
NS = 16
NW = NC * NS

EPT = N_EDGES // NW
DEG_ITERS = EPT // 16

K = 64
CH = 1024
NCH = 10
PAD_EPT = NCH * CH
PAD_EDGES = PAD_EPT * NW

LC = 2944
NBAT = LC // K
ACC_ROWS = 2560
DUMMY_DST = 2544
PQ = (0, 2504, 5008, 7512, 10000)

HST = 624
OCP = 152

_mesh = plsc.VectorSubcoreMesh(core_axis_name="c", subcore_axis_name="s")
_sc_params = pltpu.CompilerParams(needs_layout_passes=False)


@functools.partial(
    pl.kernel,
    out_type=jax.ShapeDtypeStruct((NW * N_NODES,), jnp.float32),
    mesh=_mesh,
    compiler_params=_sc_params,
    scratch_types=[
        pltpu.VMEM((EPT,), jnp.int32),
        pltpu.VMEM((N_NODES,), jnp.float32),
    ],
)
def _deg(src_hbm, out_hbm, src_v, hist_v):
    c = lax.axis_index("c")
    s = lax.axis_index("s")
    wid = s * NC + c
    pltpu.sync_copy(src_hbm.at[pl.ds(wid * EPT, EPT)], src_v)

    def _zero(i, carry):
        hist_v[pl.ds(i * 16, 16)] = jnp.zeros((16,), jnp.float32)
        return carry

    lax.fori_loop(0, N_NODES // 16, _zero, 0)

    ones = jnp.ones((16,), jnp.float32)

    def _accum(i, carry):
        idx = src_v[pl.ds(i * 16, 16)]
        plsc.addupdate_scatter(hist_v, [idx], ones)
        return carry

    lax.fori_loop(0, DEG_ITERS, _accum, 0)
    pltpu.sync_copy(hist_v, out_hbm.at[pl.ds(wid * N_NODES, N_NODES)])


def _prescale_body(pt_ref, feat_ref, h_ref, norm_ref):
    deg = jnp.sum(pt_ref[...], axis=1, keepdims=True)
    norm = jnp.where(deg > 0.0, lax.rsqrt(jnp.maximum(deg, 1e-12)), 0.0)
    norm_ref[...] = norm
    h_ref[...] = feat_ref[...] * norm


_prescale = pl.pallas_call(
    _prescale_body,
    out_shape=(
        jax.ShapeDtypeStruct((N_NODES, D_FEAT), jnp.float32),
        jax.ShapeDtypeStruct((N_NODES, 1), jnp.float32),
    ),
)


@functools.partial(
    pl.kernel,
    out_type=jax.ShapeDtypeStruct((NC, N_NODES, D_FEAT), jnp.float32),
    mesh=_mesh,
    compiler_params=_sc_params,
    scratch_types=[
        pltpu.VMEM((CH,), jnp.int32),
        pltpu.VMEM((CH,), jnp.int32),
        pltpu.VMEM((CH,), jnp.int32),
        pltpu.VMEM((CH,), jnp.int32),
        pltpu.VMEM((LC,), jnp.int32),
        pltpu.VMEM((LC,), jnp.int32),
        pltpu.VMEM((K,), jnp.int32),
        pltpu.VMEM((K,), jnp.int32),
        pltpu.VMEM((K, D_FEAT), jnp.float32),
        pltpu.VMEM((K, D_FEAT), jnp.float32),
        pltpu.VMEM_SHARED((N_NODES, D_FEAT), jnp.float32),
        pltpu.VMEM_SHARED((ACC_ROWS, D_FEAT), jnp.float32),
        pltpu.SemaphoreType.DMA,
        pltpu.SemaphoreType.DMA,
        pltpu.SemaphoreType.DMA,
        pltpu.SemaphoreType.DMA,
    ],
)
def _agg(h_hbm, src_hbm, dst_hbm, out_hbm, chs0, chs1, chd0, chd1,
         cs, cd, dbuf0, dbuf1, buf, buf1, h_sp, acc, sem, sem1,
         csem0, csem1):
    c = lax.axis_index("c")
    s = lax.axis_index("s")
    wid = s * NC + c
    chs = (chs0, chs1)
    chd = (chd0, chd1)
    csem = (csem0, csem1)

    pltpu.sync_copy(h_hbm.at[pl.ds(s * HST, HST)], h_sp.at[pl.ds(s * HST, HST)])

    @pl.when(s == NS - 1)
    def _hst_tail():
        pltpu.sync_copy(h_hbm.at[pl.ds(NS * HST, N_NODES - NS * HST)],
                        h_sp.at[pl.ds(NS * HST, N_NODES - NS * HST)])

    ebase = wid * PAD_EPT

    def _chunk_start(ch, q):
        pltpu.async_copy(src_hbm.at[pl.ds(ebase + ch * CH, CH)], chs[q],
                         csem[q])
        pltpu.async_copy(dst_hbm.at[pl.ds(ebase + ch * CH, CH)], chd[q],
                         csem[q])

    def _chunk_wait(q):
        pltpu.make_async_copy(src_hbm.at[pl.ds(0, CH)], chs[q],
                              csem[q]).wait()
        pltpu.make_async_copy(dst_hbm.at[pl.ds(0, CH)], chd[q],
                              csem[q]).wait()

    for p in range(4):
        lo = PQ[p]
        hi = PQ[p + 1]
        psz = hi - lo

        def _zbuf(i, carry):
            for j in range(D_FEAT // 16):
                buf[i, pl.ds(j * 16, 16)] = jnp.zeros((16,), jnp.float32)
            return carry

        lax.fori_loop(0, K, _zbuf, 0)
        zb = s * (ACC_ROWS // NS)
        for z in range(ACC_ROWS // NS // K):
            pltpu.sync_copy(buf, acc.at[pl.ds(zb + z * K, K)])
        zt = ACC_ROWS // NS - (ACC_ROWS // NS // K) * K
        pltpu.sync_copy(buf.at[pl.ds(0, zt)],
                        acc.at[pl.ds(zb + (ACC_ROWS // NS // K) * K, zt)])

        def _fill(i, carry):
            cs[pl.ds(i * 16, 16)] = jnp.zeros((16,), jnp.int32)
            cd[pl.ds(i * 16, 16)] = jnp.full((16,), DUMMY_DST, jnp.int32)
            return carry

        lax.fori_loop(0, LC // 16, _fill, 0)
        plsc.subcore_barrier()

        _chunk_start(0, 0)
        ptr = jnp.int32(0)
        for ch in range(NCH):
            q = ch % 2
            _chunk_wait(q)
            if ch + 1 < NCH:
                _chunk_start(ch + 1, (ch + 1) % 2)

            def _group(g, ptr_c):
                sv = chs[q][pl.ds(g * 16, 16)]
                dv = chd[q][pl.ds(g * 16, 16)]
                m = (dv >= lo) & (dv < hi)
                mi = m.astype(jnp.int32)
                pos = ptr_c + plsc.cumsum(mi) - 1
                plsc.store_scatter(cs, [pos], sv, mask=m)
                plsc.store_scatter(cd, [pos], dv - lo, mask=m)
                return ptr_c + jnp.sum(mi)

            ptr = lax.fori_loop(0, CH // 16, _group, ptr)

        del ptr
        bufs = (buf, buf1)
        sems = (sem, sem1)
        dbufs = (dbuf0, dbuf1)

        def _didx(j, b):
            for i in range(K // 16):
                dbufs[b][pl.ds(i * 16, 16)] = cd[pl.ds(j * K + i * 16, 16)]

        def _gather(j, b):
            pltpu.async_copy(h_sp.at[cs.at[pl.ds(j * K, K)]], bufs[b],
                             sems[b])

        def _gwait(b):
            pltpu.make_async_copy(h_sp.at[cs.at[pl.ds(0, K)]], bufs[b],
                                  sems[b]).wait()

        def _scatter(b):
            pltpu.sync_copy(bufs[b], acc.at[dbufs[b]], add=True)

        for b in range(2):
            _didx(b, b)
            _gather(b, b)

        def _pairs(g, carry):
            j = g * 2
            for b in range(2):
                _gwait(b)
                _scatter(b)
                _didx(j + b + 2, b)
                _gather(j + b + 2, b)
            return carry

        lax.fori_loop(0, (NBAT - 2) // 2, _pairs, 0)
        for b in range(2):
            _gwait(b)
            _scatter(b)
        plsc.subcore_barrier()

        pltpu.sync_copy(acc.at[pl.ds(s * OCP, OCP)],
                        out_hbm.at[c, pl.ds(lo + s * OCP, OCP)])

        @pl.when(s == NS - 1)
        def _ocp_tail():
            tail = psz - NS * OCP
            pltpu.sync_copy(acc.at[pl.ds(NS * OCP, tail)],
                            out_hbm.at[c, pl.ds(lo + NS * OCP, tail)])

        plsc.subcore_barrier()


def _combine_body(p_ref, norm_ref, o_ref):
    o_ref[...] = (p_ref[0] + p_ref[1]) * norm_ref[...]


_combine = pl.pallas_call(
    _combine_body,
    out_shape=jax.ShapeDtypeStruct((N_NODES, D_FEAT), jnp.float32),
)


def kernel(features, edge_index):
    src = edge_index[0]
    dst = edge_index[1]

    partials = _deg(src).reshape(NW, N_NODES)
    h, norm = _prescale(partials.T, features)

    pad = PAD_EDGES - N_EDGES
    src_p = jnp.concatenate([src, jnp.zeros((pad,), jnp.int32)])
    dst_p = jnp.concatenate([dst, jnp.full((pad,), N_NODES, jnp.int32)])

    p2 = _agg(h, src_p, dst_p)
    return _combine(p2, norm)


# speedup vs baseline: 1.2062x; 1.0684x over previous
import functools

import jax
import jax.numpy as jnp
from jax import lax
from jax.experimental import pallas as pl
from jax.experimental.pallas import tpu as pltpu
from jax.experimental.pallas import tpu_sc as plsc

N_NODES = 10000
N_EDGES = 320000
D_FEAT = 128

NC = 2
NS = 16
NW = NC * NS

EPT = N_EDGES // NW
DEG_ITERS = EPT // 16

K = 64
CH = 1024
NCH = 10
PAD_EPT = NCH * CH
PAD_EDGES = PAD_EPT * NW

LC = 2944
NBAT = LC // K
ACC_ROWS = 2560
DUMMY_DST = 2544
PQ = (0, 2504, 5008, 7512, 10000)

HST = 624
OCP = 152

_mesh = plsc.VectorSubcoreMesh(core_axis_name="c", subcore_axis_name="s")
_sc_params = pltpu.CompilerParams(needs_layout_passes=False)


@functools.partial(
    pl.kernel,
    out_type=jax.ShapeDtypeStruct((NW * N_NODES,), jnp.float32),
    mesh=_mesh,
    compiler_params=_sc_params,
    scratch_types=[
        pltpu.VMEM((EPT,), jnp.int32),
        pltpu.VMEM((N_NODES,), jnp.float32),
    ],
)
def _deg(src_hbm, out_hbm, src_v, hist_v):
    c = lax.axis_index("c")
    s = lax.axis_index("s")
    wid = s * NC + c
    pltpu.sync_copy(src_hbm.at[pl.ds(wid * EPT, EPT)], src_v)

    def _zero(i, carry):
        hist_v[pl.ds(i * 16, 16)] = jnp.zeros((16,), jnp.float32)
        return carry

    lax.fori_loop(0, N_NODES // 16, _zero, 0)

    ones = jnp.ones((16,), jnp.float32)

    def _accum(i, carry):
        idx = src_v[pl.ds(i * 16, 16)]
        plsc.addupdate_scatter(hist_v, [idx], ones)
        return carry

    lax.fori_loop(0, DEG_ITERS, _accum, 0)
    pltpu.sync_copy(hist_v, out_hbm.at[pl.ds(wid * N_NODES, N_NODES)])


def _prescale_body(pt_ref, feat_ref, h_ref, norm_ref):
    deg = jnp.sum(pt_ref[...], axis=1, keepdims=True)
    norm = jnp.where(deg > 0.0, lax.rsqrt(jnp.maximum(deg, 1e-12)), 0.0)
    norm_ref[...] = norm
    h_ref[...] = feat_ref[...] * norm


_prescale = pl.pallas_call(
    _prescale_body,
    out_shape=(
        jax.ShapeDtypeStruct((N_NODES, D_FEAT), jnp.float32),
        jax.ShapeDtypeStruct((N_NODES, 1), jnp.float32),
    ),
)


@functools.partial(
    pl.kernel,
    out_type=jax.ShapeDtypeStruct((NC, N_NODES, D_FEAT), jnp.float32),
    mesh=_mesh,
    compiler_params=_sc_params,
    scratch_types=[
        pltpu.VMEM((CH,), jnp.int32),
        pltpu.VMEM((CH,), jnp.int32),
        pltpu.VMEM((CH,), jnp.int32),
        pltpu.VMEM((CH,), jnp.int32),
        pltpu.VMEM((LC,), jnp.int32),
        pltpu.VMEM((LC,), jnp.int32),
        pltpu.VMEM((K,), jnp.int32),
        pltpu.VMEM((K,), jnp.int32),
        pltpu.VMEM((K, D_FEAT), jnp.float32),
        pltpu.VMEM((K, D_FEAT), jnp.float32),
        pltpu.VMEM_SHARED((N_NODES, D_FEAT), jnp.float32),
        pltpu.VMEM_SHARED((ACC_ROWS, D_FEAT), jnp.float32),
        pltpu.SemaphoreType.DMA,
        pltpu.SemaphoreType.DMA,
        pltpu.SemaphoreType.DMA,
        pltpu.SemaphoreType.DMA,
    ],
)
def _agg(h_hbm, src_hbm, dst_hbm, out_hbm, chs0, chs1, chd0, chd1,
         cs, cd, dbuf0, dbuf1, buf, buf1, h_sp, acc, sem, sem1,
         csem0, csem1):
    c = lax.axis_index("c")
    s = lax.axis_index("s")
    wid = s * NC + c
    chs = (chs0, chs1)
    chd = (chd0, chd1)
    csem = (csem0, csem1)

    pltpu.sync_copy(h_hbm.at[pl.ds(s * HST, HST)], h_sp.at[pl.ds(s * HST, HST)])

    @pl.when(s == NS - 1)
    def _hst_tail():
        pltpu.sync_copy(h_hbm.at[pl.ds(NS * HST, N_NODES - NS * HST)],
                        h_sp.at[pl.ds(NS * HST, N_NODES - NS * HST)])

    ebase = wid * PAD_EPT

    def _chunk_start(ch, q):
        pltpu.async_copy(src_hbm.at[pl.ds(ebase + ch * CH, CH)], chs[q],
                         csem[q])
        pltpu.async_copy(dst_hbm.at[pl.ds(ebase + ch * CH, CH)], chd[q],
                         csem[q])

    def _chunk_wait(q):
        pltpu.make_async_copy(src_hbm.at[pl.ds(0, CH)], chs[q],
                              csem[q]).wait()
        pltpu.make_async_copy(dst_hbm.at[pl.ds(0, CH)], chd[q],
                              csem[q]).wait()

    for p in range(4):
        lo = PQ[p]
        hi = PQ[p + 1]
        psz = hi - lo

        def _zbuf(i, carry):
            for j in range(D_FEAT // 16):
                buf[i, pl.ds(j * 16, 16)] = jnp.zeros((16,), jnp.float32)
            return carry

        lax.fori_loop(0, K, _zbuf, 0)
        zb = s * (ACC_ROWS // NS)
        for z in range(ACC_ROWS // NS // K):
            pltpu.sync_copy(buf, acc.at[pl.ds(zb + z * K, K)])
        zt = ACC_ROWS // NS - (ACC_ROWS // NS // K) * K
        pltpu.sync_copy(buf.at[pl.ds(0, zt)],
                        acc.at[pl.ds(zb + (ACC_ROWS // NS // K) * K, zt)])

        def _fill(i, carry):
            cs[pl.ds(i * 16, 16)] = jnp.zeros((16,), jnp.int32)
            cd[pl.ds(i * 16, 16)] = jnp.full((16,), DUMMY_DST, jnp.int32)
            return carry

        lax.fori_loop(0, LC // 16, _fill, 0)
        plsc.subcore_barrier()

        _chunk_start(0, 0)
        ptr = jnp.int32(0)
        for ch in range(NCH):
            q = ch % 2
            _chunk_wait(q)
            if ch + 1 < NCH:
                _chunk_start(ch + 1, (ch + 1) % 2)

            def _group(g, ptr_c):
                sv = chs[q][pl.ds(g * 16, 16)]
                dv = chd[q][pl.ds(g * 16, 16)]
                m = (dv >= lo) & (dv < hi)
                mi = m.astype(jnp.int32)
                pos = ptr_c + plsc.cumsum(mi) - 1
                plsc.store_scatter(cs, [pos], sv, mask=m)
                plsc.store_scatter(cd, [pos], dv - lo, mask=m)
                return ptr_c + jnp.sum(mi)

            ptr = lax.fori_loop(0, CH // 16, _group, ptr)

        bufs = (buf, buf1)
        sems = (sem, sem1)
        dbufs = (dbuf0, dbuf1)

        def _didx(j, b):
            for i in range(K // 16):
                dbufs[b][pl.ds(i * 16, 16)] = cd[pl.ds(j * K + i * 16, 16)]

        def _gather(j, b):
            pltpu.async_copy(h_sp.at[cs.at[pl.ds(j * K, K)]], bufs[b],
                             sems[b])

        def _gwait(b):
            pltpu.make_async_copy(h_sp.at[cs.at[pl.ds(0, K)]], bufs[b],
                                  sems[b]).wait()

        def _scatter(b):
            pltpu.sync_copy(bufs[b], acc.at[dbufs[b]], add=True)

        for b in range(2):
            _didx(b, b)
            _gather(b, b)

        def _pairs(g, carry):
            j = g * 2
            for b in range(2):
                _gwait(b)
                _scatter(b)
                _didx(j + b + 2, b)
                _gather(j + b + 2, b)
            return carry

        npair = (ptr + 2 * K - 1) // (2 * K)
        lax.fori_loop(0, npair - 1, _pairs, 0)
        for b in range(2):
            _gwait(b)
            _scatter(b)
        plsc.subcore_barrier()

        pltpu.sync_copy(acc.at[pl.ds(s * OCP, OCP)],
                        out_hbm.at[c, pl.ds(lo + s * OCP, OCP)])

        @pl.when(s == NS - 1)
        def _ocp_tail():
            tail = psz - NS * OCP
            pltpu.sync_copy(acc.at[pl.ds(NS * OCP, tail)],
                            out_hbm.at[c, pl.ds(lo + NS * OCP, tail)])

        plsc.subcore_barrier()


def _combine_body(p_ref, norm_ref, o_ref):
    o_ref[...] = (p_ref[0] + p_ref[1]) * norm_ref[...]


_combine = pl.pallas_call(
    _combine_body,
    out_shape=jax.ShapeDtypeStruct((N_NODES, D_FEAT), jnp.float32),
)


def kernel(features, edge_index):
    src = edge_index[0]
    dst = edge_index[1]

    partials = _deg(src).reshape(NW, N_NODES)
    h, norm = _prescale(partials.T, features)

    pad = PAD_EDGES - N_EDGES
    src_p = jnp.concatenate([src, jnp.zeros((pad,), jnp.int32)])
    dst_p = jnp.concatenate([dst, jnp.full((pad,), N_NODES, jnp.int32)])

    p2 = _agg(h, src_p, dst_p)
    return _combine(p2, norm)
